# baseline (speedup 1.0000x reference)
import jax
import jax.numpy as jnp
from jax.experimental import pallas as pl

N = 10000
B = 64
STEPS = 8


def kernel(node_features, edge_features, Esrc, Etgt, batch, W1, b1, W2, b2, W3, b3,
           ee1_Wa, ee1_ba, ee1_Wb, ee1_bb,
           ee2_Wa, ee2_ba, ee2_Wb, ee2_bb,
           ee3_Wa, ee3_ba, ee3_Wb, ee3_bb,
           lstm_Wih, lstm_Whh, lstm_bih, lstm_bhh):
    n = node_features.shape[0]
    bs = B

    def ee(ef, Wa, ba, Wb, bb):
        h = jax.nn.relu(ef @ Wa + ba)
        return h @ Wb + bb

    def gc(x, W, b, ef):
        support = x @ W
        msg = support[Esrc] * ef
        agg = jax.ops.segment_sum(msg, Etgt, num_segments=n)
        return agg + b

    ef1 = ee(edge_features, ee1_Wa, ee1_ba, ee1_Wb, ee1_bb)
    x = jax.nn.relu(gc(node_features, W1, b1, ef1))
    ef2 = ee(edge_features, ee2_Wa, ee2_ba, ee2_Wb, ee2_bb)
    x = jax.nn.relu(gc(x, W2, b2, ef2))
    ef3 = ee(edge_features, ee3_Wa, ee3_ba, ee3_Wb, ee3_bb)
    x = gc(x, W3, b3, ef3)

    d = x.shape[1]
    h = jnp.zeros((bs, d), jnp.float32)
    c = jnp.zeros((bs, d), jnp.float32)
    q_star = jnp.zeros((bs, 2 * d), jnp.float32)
    for _ in range(STEPS):
        gates = q_star @ lstm_Wih.T + lstm_bih + h @ lstm_Whh.T + lstm_bhh
        i, f, g, o = jnp.split(gates, 4, axis=1)
        i = jax.nn.sigmoid(i); f = jax.nn.sigmoid(f)
        g = jnp.tanh(g); o = jax.nn.sigmoid(o)
        c = f * c + i * g
        h = o * jnp.tanh(c)
        q = h
        e = jnp.sum(x * q[batch], axis=1, keepdims=True)
        emax = jax.ops.segment_max(e, batch, num_segments=bs)
        ex = jnp.exp(e - emax[batch])
        denom = jax.ops.segment_sum(ex, batch, num_segments=bs)
        a = ex / (denom[batch] + 1e-16)
        r = jax.ops.segment_sum(a * x, batch, num_segments=bs)
        q_star = jnp.concatenate([q, r], axis=1)

    out = pl.pallas_call(
        lambda x_ref, o_ref: o_ref.__setitem__(slice(None), x_ref[...]),
        out_shape=jax.ShapeDtypeStruct(q_star[:, :d].shape, jnp.float32),
    )(q_star[:, :d])
    return out


# speedup vs baseline: 1.3872x; 1.3872x over previous
import functools

import jax
import jax.numpy as jnp
from jax import lax
from jax.experimental import pallas as pl
from jax.experimental.pallas import tpu as pltpu

N = 10000
E = 320000
DF = 128
DE = 16
DH = 256
DO = 128
B = 64
STEPS = 8

NPAD = 10496
EK = 1024
E_PAD = 327680
EB = E_PAD // EK
R = 256


def _mm(x, W, bias=None, relu=False):
    n, din = x.shape
    dout = W.shape[1]
    rb = 256
    grid = n // rb

    def body(x_ref, w_ref, b_ref, o_ref):
        xv = x_ref[...]
        if relu:
            xv = jnp.maximum(xv + b_ref[...], 0.0)
        o_ref[...] = jnp.dot(xv, w_ref[...], preferred_element_type=jnp.float32)

    if bias is None:
        bias = jnp.zeros((1, din), jnp.float32)
    return pl.pallas_call(
        body,
        grid=(grid,),
        in_specs=[
            pl.BlockSpec((rb, din), lambda i: (i, 0)),
            pl.BlockSpec((din, dout), lambda i: (0, 0)),
            pl.BlockSpec((1, din), lambda i: (0, 0)),
        ],
        out_specs=pl.BlockSpec((rb, dout), lambda i: (i, 0)),
        out_shape=jax.ShapeDtypeStruct((n, dout), jnp.float32),
    )(x, W, bias)


def _edge_layer(ef_s, gathered, et3, Wa, ba, Wb, bb, dout):
    def body(ef_ref, g_ref, et_s_ref, et_v_ref, wa_ref, ba_ref, wb_ref,
             bb_ref, o_ref):
        i = pl.program_id(0)

        @pl.when(i == 0)
        def _init():
            o_ref[...] = jnp.zeros_like(o_ref)

        h = jnp.dot(ef_ref[...], wa_ref[...],
                    preferred_element_type=jnp.float32) + ba_ref[...]
        h = jnp.maximum(h, 0.0)
        efv = jnp.dot(h, wb_ref[...],
                      preferred_element_type=jnp.float32) + bb_ref[...]
        msg = g_ref[...] * efv

        t_row = et_v_ref[0]
        t0 = et_s_ref[0, 0, 0]
        tl = et_s_ref[0, 0, EK - 1]
        base0 = (t0 // 8) * 8
        nch = (tl - base0) // R + 1

        def chunk(c, carry):
            base = base0 + c * R
            rows = lax.broadcasted_iota(jnp.int32, (R, EK), 0) + base
            oht = (rows == t_row).astype(jnp.float32)
            part = jnp.dot(oht, msg, preferred_element_type=jnp.float32)
            o_ref[pl.ds(base, R), :] += part
            return carry

        lax.fori_loop(0, nch, chunk, 0)

    dh = Wa.shape[1]
    return pl.pallas_call(
        body,
        grid=(EB,),
        in_specs=[
            pl.BlockSpec((EK, DE), lambda i: (i, 0)),
            pl.BlockSpec((EK, dout), lambda i: (i, 0)),
            pl.BlockSpec((1, 1, EK), lambda i: (i, 0, 0),
                         memory_space=pltpu.SMEM),
            pl.BlockSpec((1, 1, EK), lambda i: (i, 0, 0)),
            pl.BlockSpec((DE, dh), lambda i: (0, 0)),
            pl.BlockSpec((1, dh), lambda i: (0, 0)),
            pl.BlockSpec((dh, dout), lambda i: (0, 0)),
            pl.BlockSpec((1, dout), lambda i: (0, 0)),
        ],
        out_specs=pl.BlockSpec((NPAD, dout), lambda i: (0, 0)),
        out_shape=jax.ShapeDtypeStruct((NPAD, dout), jnp.float32),
    )(ef_s, gathered, et3, et3, Wa, ba.reshape(1, dh), Wb, bb.reshape(1, dout))


SC_CH = 80
SC_W = 32
SC_PER_W = E_PAD // SC_W
SC_NCH = SC_PER_W // SC_CH


def _sc_gather(table, idx):
    from jax.experimental.pallas import tpu_sc as plsc

    d = table.shape[1]
    mesh = plsc.VectorSubcoreMesh(core_axis_name="c", subcore_axis_name="s")

    @functools.partial(
        pl.kernel,
        out_type=jax.ShapeDtypeStruct((E_PAD, d), jnp.float32),
        mesh=mesh,
        scratch_types=[
            pltpu.VMEM((2, SC_CH), jnp.int32),
            pltpu.VMEM((2, SC_CH, d), jnp.float32),
            pltpu.SemaphoreType.DMA((2,)),
        ],
    )
    def k(table_hbm, idx_hbm, out_hbm, idx_v, rows_v, sems):
        wid = lax.axis_index("s") * 2 + lax.axis_index("c")
        base = wid * SC_PER_W

        def issue(ci, slot):
            off = base + ci * SC_CH
            pltpu.sync_copy(idx_hbm.at[pl.ds(off, SC_CH)], idx_v.at[slot])
            return pltpu.async_copy(
                table_hbm.at[idx_v.at[slot]], rows_v.at[slot], sems.at[slot])

        issue(0, 0)

        def body(ci, carry):
            slot = lax.rem(ci, 2)
            nslot = lax.rem(ci + 1, 2)

            @pl.when(ci + 1 < SC_NCH)
            def _():
                issue(ci + 1, nslot)

            pltpu.make_async_copy(
                table_hbm.at[idx_v.at[slot]], rows_v.at[slot],
                sems.at[slot]).wait()
            pltpu.sync_copy(rows_v.at[slot],
                            out_hbm.at[pl.ds(base + ci * SC_CH, SC_CH)])
            return carry

        lax.fori_loop(0, SC_NCH, body, 0)

    return k(table, idx)


def _set2set(agg3, b3, batch_col, batch_row, Wih, Whh, bih, bhh):
    def body(x_ref, b3_ref, bc_ref, br_ref, wih_ref, whh_ref, bih_ref,
             bhh_ref, o_ref):
        rowid = lax.broadcasted_iota(jnp.int32, (NPAD, DO), 0)
        x = jnp.where(rowid < N, x_ref[...] + b3_ref[...], 0.0)

        oh = (bc_ref[...] == lax.broadcasted_iota(
            jnp.int32, (NPAD, B), 1)).astype(jnp.float32)
        ohT = (br_ref[...] == lax.broadcasted_iota(
            jnp.int32, (B, NPAD), 0)).astype(jnp.float32)

        wih = wih_ref[...]
        whh = whh_ref[...]
        bih = bih_ref[...]
        bhh = bhh_ref[...]

        h = jnp.zeros((B, DO), jnp.float32)
        c = jnp.zeros((B, DO), jnp.float32)
        q_star = jnp.zeros((B, 2 * DO), jnp.float32)
        for _ in range(STEPS):
            gates = (lax.dot_general(q_star, wih, (((1,), (1,)), ((), ())),
                                     preferred_element_type=jnp.float32)
                     + bih
                     + lax.dot_general(h, whh, (((1,), (1,)), ((), ())),
                                       preferred_element_type=jnp.float32)
                     + bhh)
            ig = jax.nn.sigmoid(gates[:, :DO])
            fg = jax.nn.sigmoid(gates[:, DO:2 * DO])
            gg = jnp.tanh(gates[:, 2 * DO:3 * DO])
            og = jax.nn.sigmoid(gates[:, 3 * DO:])
            c = fg * c + ig * gg
            h = og * jnp.tanh(c)
            q = h

            qb = jnp.dot(oh, q, preferred_element_type=jnp.float32)
            e = jnp.sum(x * qb, axis=1, keepdims=True)
            t = jnp.where(oh > 0.5, e, -1e30)
            emax = jnp.max(t, axis=0, keepdims=True)
            sub = lax.dot_general(oh, emax, (((1,), (1,)), ((), ())),
                                  preferred_element_type=jnp.float32)
            ex = jnp.exp(e - sub)
            denom = jnp.dot(ohT, ex, preferred_element_type=jnp.float32)
            dnode = jnp.dot(oh, denom, preferred_element_type=jnp.float32)
            a = ex / (dnode + 1e-16)
            r = jnp.dot(ohT, a * x, preferred_element_type=jnp.float32)
            q_star = jnp.concatenate([q, r], axis=1)
        o_ref[...] = q_star[:, :DO]

    return pl.pallas_call(
        body,
        out_shape=jax.ShapeDtypeStruct((B, DO), jnp.float32),
    )(agg3, b3, batch_col, batch_row, Wih, Whh, bih, bhh)


def kernel(node_features, edge_features, Esrc, Etgt, batch,
           W1, b1, W2, b2, W3, b3,
           ee1_Wa, ee1_ba, ee1_Wb, ee1_bb,
           ee2_Wa, ee2_ba, ee2_Wb, ee2_bb,
           ee3_Wa, ee3_ba, ee3_Wb, ee3_bb,
           lstm_Wih, lstm_Whh, lstm_bih, lstm_bhh):
    perm = jnp.argsort(Etgt)
    et_s = jnp.concatenate(
        [Etgt[perm], jnp.full((E_PAD - E,), N, jnp.int32)])
    es_s = jnp.concatenate(
        [Esrc[perm], jnp.zeros((E_PAD - E,), jnp.int32)])
    ef_s = jnp.concatenate(
        [edge_features[perm],
         jnp.zeros((E_PAD - E, DE), jnp.float32)], axis=0)
    et3 = et_s.reshape(EB, 1, EK)

    xp = jnp.zeros((NPAD, DF), jnp.float32).at[:N].set(node_features)
    batch_col = jnp.full((NPAD, 1), B, jnp.int32).at[:N, 0].set(batch)
    batch_row = batch_col.reshape(1, NPAD)

    sup1 = _mm(xp, W1)
    g1 = jnp.take(sup1, es_s, axis=0)
    agg1 = _edge_layer(ef_s, g1, et3, ee1_Wa, ee1_ba, ee1_Wb, ee1_bb, DH)
    sup2 = _mm(agg1, W2, bias=b1.reshape(1, DH), relu=True)
    g2 = jnp.take(sup2, es_s, axis=0)
    agg2 = _edge_layer(ef_s, g2, et3, ee2_Wa, ee2_ba, ee2_Wb, ee2_bb, DH)
    sup3 = _mm(agg2, W3, bias=b2.reshape(1, DH), relu=True)
    g3 = jnp.take(sup3, es_s, axis=0)
    agg3 = _edge_layer(ef_s, g3, et3, ee3_Wa, ee3_ba, ee3_Wb, ee3_bb, DO)

    return _set2set(agg3, b3.reshape(1, DO), batch_col, batch_row,
                    lstm_Wih, lstm_Whh,
                    lstm_bih.reshape(1, 4 * DO), lstm_bhh.reshape(1, 4 * DO))


# speedup vs baseline: 2.0937x; 1.5093x over previous
import functools

import jax
import jax.numpy as jnp
from jax import lax
from jax.experimental import pallas as pl
from jax.experimental.pallas import tpu as pltpu

N = 10000
E = 320000
DF = 128
DE = 16
DH = 256
DO = 128
B = 64
STEPS = 8

NPAD = 10496
EK = 1024
E_PAD = 327680
EB = E_PAD // EK
R = 256


def _mm(x, W, bias=None, relu=False):
    n, din = x.shape
    dout = W.shape[1]
    rb = 256
    grid = n // rb

    def body(x_ref, w_ref, b_ref, o_ref):
        xv = x_ref[...]
        if relu:
            xv = jnp.maximum(xv + b_ref[...], 0.0)
        o_ref[...] = jnp.dot(xv, w_ref[...], preferred_element_type=jnp.float32)

    if bias is None:
        bias = jnp.zeros((1, din), jnp.float32)
    return pl.pallas_call(
        body,
        grid=(grid,),
        in_specs=[
            pl.BlockSpec((rb, din), lambda i: (i, 0)),
            pl.BlockSpec((din, dout), lambda i: (0, 0)),
            pl.BlockSpec((1, din), lambda i: (0, 0)),
        ],
        out_specs=pl.BlockSpec((rb, dout), lambda i: (i, 0)),
        out_shape=jax.ShapeDtypeStruct((n, dout), jnp.float32),
    )(x, W, bias)


def _edge_layer(ef_s, gathered, et3, Wa, ba, Wb, bb, dout):
    def body(ef_ref, g_ref, et_s_ref, et_v_ref, wa_ref, ba_ref, wb_ref,
             bb_ref, o_ref):
        i = pl.program_id(0)

        @pl.when(i == 0)
        def _init():
            o_ref[...] = jnp.zeros_like(o_ref)

        h = jnp.dot(ef_ref[...], wa_ref[...],
                    preferred_element_type=jnp.float32) + ba_ref[...]
        h = jnp.maximum(h, 0.0)
        efv = jnp.dot(h, wb_ref[...],
                      preferred_element_type=jnp.float32) + bb_ref[...]
        msg = g_ref[...] * efv

        t_row = et_v_ref[0]
        t0 = et_s_ref[0, 0, 0]
        tl = et_s_ref[0, 0, EK - 1]
        base0 = (t0 // 8) * 8
        nch = (tl - base0) // R + 1

        def chunk(c, carry):
            base = base0 + c * R
            rows = lax.broadcasted_iota(jnp.int32, (R, EK), 0) + base
            oht = (rows == t_row).astype(jnp.float32)
            part = jnp.dot(oht, msg, preferred_element_type=jnp.float32)
            o_ref[pl.ds(base, R), :] += part
            return carry

        lax.fori_loop(0, nch, chunk, 0)

    dh = Wa.shape[1]
    return pl.pallas_call(
        body,
        grid=(EB,),
        in_specs=[
            pl.BlockSpec((EK, DE), lambda i: (i, 0)),
            pl.BlockSpec((EK, dout), lambda i: (i, 0)),
            pl.BlockSpec((1, 1, EK), lambda i: (i, 0, 0),
                         memory_space=pltpu.SMEM),
            pl.BlockSpec((1, 1, EK), lambda i: (i, 0, 0)),
            pl.BlockSpec((DE, dh), lambda i: (0, 0)),
            pl.BlockSpec((1, dh), lambda i: (0, 0)),
            pl.BlockSpec((dh, dout), lambda i: (0, 0)),
            pl.BlockSpec((1, dout), lambda i: (0, 0)),
        ],
        out_specs=pl.BlockSpec((NPAD, dout), lambda i: (0, 0)),
        out_shape=jax.ShapeDtypeStruct((NPAD, dout), jnp.float32),
    )(ef_s, gathered, et3, et3, Wa, ba.reshape(1, dh), Wb, bb.reshape(1, dout))


SC_CH = 80
SC_W = 32
SC_PER_W = E_PAD // SC_W
SC_NCH = SC_PER_W // SC_CH


def _sc_gather(table, idx):
    from jax.experimental.pallas import tpu_sc as plsc

    d = table.shape[1]
    mesh = plsc.VectorSubcoreMesh(core_axis_name="c", subcore_axis_name="s")

    @functools.partial(
        pl.kernel,
        out_type=jax.ShapeDtypeStruct((E_PAD, d), jnp.float32),
        mesh=mesh,
        scratch_types=[
            pltpu.VMEM((2, SC_CH), jnp.int32),
            pltpu.VMEM((2, SC_CH, d), jnp.float32),
            pltpu.SemaphoreType.DMA,
            pltpu.SemaphoreType.DMA,
        ],
    )
    def k(table_hbm, idx_hbm, out_hbm, idx_v, rows_v, sem0, sem1):
        wid = lax.axis_index("s") * 2 + lax.axis_index("c")
        base = wid * SC_PER_W
        sems = (sem0, sem1)

        def issue(ci, b):
            off = base + ci * SC_CH
            pltpu.sync_copy(idx_hbm.at[pl.ds(off, SC_CH)], idx_v.at[b])
            pltpu.async_copy(table_hbm.at[idx_v.at[b]], rows_v.at[b], sems[b])

        issue(0, 0)
        issue(1, 1)

        def body(cg, carry):
            for b in range(2):
                ci = cg * 2 + b
                pltpu.make_async_copy(
                    table_hbm.at[idx_v.at[b]], rows_v.at[b], sems[b]).wait()
                pltpu.sync_copy(rows_v.at[b],
                                out_hbm.at[pl.ds(base + ci * SC_CH, SC_CH)])

                @pl.when(ci + 2 < SC_NCH)
                def _():
                    issue(ci + 2, b)
            return carry

        lax.fori_loop(0, SC_NCH // 2, body, 0)

    return k(table, idx)


def _set2set(agg3, b3, batch_col, batch_row, Wih, Whh, bih, bhh):
    def body(x_ref, b3_ref, bc_ref, br_ref, wih_ref, whh_ref, bih_ref,
             bhh_ref, o_ref):
        rowid = lax.broadcasted_iota(jnp.int32, (NPAD, DO), 0)
        x = jnp.where(rowid < N, x_ref[...] + b3_ref[...], 0.0)

        oh = (bc_ref[...] == lax.broadcasted_iota(
            jnp.int32, (NPAD, B), 1)).astype(jnp.float32)
        ohT = (br_ref[...] == lax.broadcasted_iota(
            jnp.int32, (B, NPAD), 0)).astype(jnp.float32)

        wih = wih_ref[...]
        whh = whh_ref[...]
        bih = bih_ref[...]
        bhh = bhh_ref[...]

        h = jnp.zeros((B, DO), jnp.float32)
        c = jnp.zeros((B, DO), jnp.float32)
        q_star = jnp.zeros((B, 2 * DO), jnp.float32)
        for _ in range(STEPS):
            gates = (lax.dot_general(q_star, wih, (((1,), (1,)), ((), ())),
                                     preferred_element_type=jnp.float32)
                     + bih
                     + lax.dot_general(h, whh, (((1,), (1,)), ((), ())),
                                       preferred_element_type=jnp.float32)
                     + bhh)
            ig = jax.nn.sigmoid(gates[:, :DO])
            fg = jax.nn.sigmoid(gates[:, DO:2 * DO])
            gg = jnp.tanh(gates[:, 2 * DO:3 * DO])
            og = jax.nn.sigmoid(gates[:, 3 * DO:])
            c = fg * c + ig * gg
            h = og * jnp.tanh(c)
            q = h

            qb = jnp.dot(oh, q, preferred_element_type=jnp.float32)
            e = jnp.sum(x * qb, axis=1, keepdims=True)
            t = jnp.where(oh > 0.5, e, -1e30)
            emax = jnp.max(t, axis=0, keepdims=True)
            sub = lax.dot_general(oh, emax, (((1,), (1,)), ((), ())),
                                  preferred_element_type=jnp.float32)
            ex = jnp.exp(e - sub)
            denom = jnp.dot(ohT, ex, preferred_element_type=jnp.float32)
            dnode = jnp.dot(oh, denom, preferred_element_type=jnp.float32)
            a = ex / (dnode + 1e-16)
            r = jnp.dot(ohT, a * x, preferred_element_type=jnp.float32)
            q_star = jnp.concatenate([q, r], axis=1)
        o_ref[...] = q_star[:, :DO]

    return pl.pallas_call(
        body,
        out_shape=jax.ShapeDtypeStruct((B, DO), jnp.float32),
    )(agg3, b3, batch_col, batch_row, Wih, Whh, bih, bhh)


def kernel(node_features, edge_features, Esrc, Etgt, batch,
           W1, b1, W2, b2, W3, b3,
           ee1_Wa, ee1_ba, ee1_Wb, ee1_bb,
           ee2_Wa, ee2_ba, ee2_Wb, ee2_bb,
           ee3_Wa, ee3_ba, ee3_Wb, ee3_bb,
           lstm_Wih, lstm_Whh, lstm_bih, lstm_bhh):
    perm = jnp.argsort(Etgt)
    et_s = jnp.concatenate(
        [Etgt[perm], jnp.full((E_PAD - E,), N, jnp.int32)])
    es_s = jnp.concatenate(
        [Esrc[perm], jnp.zeros((E_PAD - E,), jnp.int32)])
    ef_s = jnp.concatenate(
        [edge_features[perm],
         jnp.zeros((E_PAD - E, DE), jnp.float32)], axis=0)
    et3 = et_s.reshape(EB, 1, EK)

    xp = jnp.zeros((NPAD, DF), jnp.float32).at[:N].set(node_features)
    batch_col = jnp.full((NPAD, 1), B, jnp.int32).at[:N, 0].set(batch)
    batch_row = batch_col.reshape(1, NPAD)

    sup1 = _mm(xp, W1)
    g1 = _sc_gather(sup1, es_s)
    agg1 = _edge_layer(ef_s, g1, et3, ee1_Wa, ee1_ba, ee1_Wb, ee1_bb, DH)
    sup2 = _mm(agg1, W2, bias=b1.reshape(1, DH), relu=True)
    g2 = _sc_gather(sup2, es_s)
    agg2 = _edge_layer(ef_s, g2, et3, ee2_Wa, ee2_ba, ee2_Wb, ee2_bb, DH)
    sup3 = _mm(agg2, W3, bias=b2.reshape(1, DH), relu=True)
    g3 = _sc_gather(sup3, es_s)
    agg3 = _edge_layer(ef_s, g3, et3, ee3_Wa, ee3_ba, ee3_Wb, ee3_bb, DO)

    return _set2set(agg3, b3.reshape(1, DO), batch_col, batch_row,
                    lstm_Wih, lstm_Whh,
                    lstm_bih.reshape(1, 4 * DO), lstm_bhh.reshape(1, 4 * DO))


# speedup vs baseline: 2.1300x; 1.0173x over previous
import functools

import jax
import jax.numpy as jnp
from jax import lax
from jax.experimental import pallas as pl
from jax.experimental.pallas import tpu as pltpu

N = 10000
E = 320000
DF = 128
DE = 16
DH = 256
DO = 128
B = 64
STEPS = 8

NPAD = 10496
EK = 1024
E_PAD = 327680
EB = E_PAD // EK
R = 256


def _mm(x, W, bias=None, relu=False):
    n, din = x.shape
    dout = W.shape[1]
    rb = 256
    grid = n // rb

    def body(x_ref, w_ref, b_ref, o_ref):
        xv = x_ref[...]
        if relu:
            xv = jnp.maximum(xv + b_ref[...], 0.0)
        o_ref[...] = jnp.dot(xv, w_ref[...], preferred_element_type=jnp.float32)

    if bias is None:
        bias = jnp.zeros((1, din), jnp.float32)
    return pl.pallas_call(
        body,
        grid=(grid,),
        in_specs=[
            pl.BlockSpec((rb, din), lambda i: (i, 0)),
            pl.BlockSpec((din, dout), lambda i: (0, 0)),
            pl.BlockSpec((1, din), lambda i: (0, 0)),
        ],
        out_specs=pl.BlockSpec((rb, dout), lambda i: (i, 0)),
        out_shape=jax.ShapeDtypeStruct((n, dout), jnp.float32),
    )(x, W, bias)


def _edge_layer(ef_s, gathered, et3, Wa, ba, Wb, bb, dout):
    def body(ef_ref, g_ref, et_s_ref, et_v_ref, wa_ref, ba_ref, wb_ref,
             bb_ref, o_ref):
        i = pl.program_id(0)

        @pl.when(i == 0)
        def _init():
            o_ref[...] = jnp.zeros_like(o_ref)

        h = jnp.dot(ef_ref[...], wa_ref[...],
                    preferred_element_type=jnp.float32) + ba_ref[...]
        h = jnp.maximum(h, 0.0)
        efv = jnp.dot(h, wb_ref[...],
                      preferred_element_type=jnp.float32) + bb_ref[...]
        msg = g_ref[...] * efv

        t_row = et_v_ref[0]
        t0 = et_s_ref[0, 0, 0]
        tl = et_s_ref[0, 0, EK - 1]
        base0 = (t0 // 8) * 8
        nch = (tl - base0) // R + 1

        def chunk(c, carry):
            base = base0 + c * R
            rows = lax.broadcasted_iota(jnp.int32, (R, EK), 0) + base
            oht = (rows == t_row).astype(jnp.float32)
            part = jnp.dot(oht, msg, preferred_element_type=jnp.float32)
            o_ref[pl.ds(base, R), :] += part
            return carry

        lax.fori_loop(0, nch, chunk, 0)

    dh = Wa.shape[1]
    return pl.pallas_call(
        body,
        grid=(EB,),
        in_specs=[
            pl.BlockSpec((EK, DE), lambda i: (i, 0)),
            pl.BlockSpec((EK, dout), lambda i: (i, 0)),
            pl.BlockSpec((1, 1, EK), lambda i: (i, 0, 0),
                         memory_space=pltpu.SMEM),
            pl.BlockSpec((1, 1, EK), lambda i: (i, 0, 0)),
            pl.BlockSpec((DE, dh), lambda i: (0, 0)),
            pl.BlockSpec((1, dh), lambda i: (0, 0)),
            pl.BlockSpec((dh, dout), lambda i: (0, 0)),
            pl.BlockSpec((1, dout), lambda i: (0, 0)),
        ],
        out_specs=pl.BlockSpec((NPAD, dout), lambda i: (0, 0)),
        out_shape=jax.ShapeDtypeStruct((NPAD, dout), jnp.float32),
    )(ef_s, gathered, et3, et3, Wa, ba.reshape(1, dh), Wb, bb.reshape(1, dout))


SC_CH = 128
SC_W = 32
SC_PER_W = E_PAD // SC_W
SC_NCH = SC_PER_W // SC_CH


def _sc_gather(table, idx):
    from jax.experimental.pallas import tpu_sc as plsc

    d = table.shape[1]
    mesh = plsc.VectorSubcoreMesh(core_axis_name="c", subcore_axis_name="s")

    @functools.partial(
        pl.kernel,
        out_type=jax.ShapeDtypeStruct((E_PAD, d), jnp.float32),
        mesh=mesh,
        scratch_types=[
            pltpu.VMEM((SC_PER_W,), jnp.int32),
            pltpu.VMEM((2, SC_CH, d), jnp.float32),
            pltpu.SemaphoreType.DMA,
            pltpu.SemaphoreType.DMA,
        ],
    )
    def k(table_hbm, idx_hbm, out_hbm, idx_all, rows_v, sem0, sem1):
        wid = lax.axis_index("s") * 2 + lax.axis_index("c")
        base = wid * SC_PER_W
        sems = (sem0, sem1)

        pltpu.sync_copy(idx_hbm.at[pl.ds(base, SC_PER_W)], idx_all)

        def issue(ci, b):
            pltpu.async_copy(
                table_hbm.at[idx_all.at[pl.ds(ci * SC_CH, SC_CH)]],
                rows_v.at[b], sems[b])

        issue(0, 0)

        def body(cg, carry):
            for b in range(2):
                ci = cg * 2 + b

                @pl.when(ci + 1 < SC_NCH)
                def _():
                    issue(ci + 1, 1 - b)

                pltpu.make_async_copy(
                    table_hbm.at[idx_all.at[pl.ds(ci * SC_CH, SC_CH)]],
                    rows_v.at[b], sems[b]).wait()
                pltpu.sync_copy(rows_v.at[b],
                                out_hbm.at[pl.ds(base + ci * SC_CH, SC_CH)])
            return carry

        lax.fori_loop(0, SC_NCH // 2, body, 0)

    return k(table, idx)


def _set2set(agg3, b3, batch_col, batch_row, Wih, Whh, bih, bhh):
    def body(x_ref, b3_ref, bc_ref, br_ref, wih_ref, whh_ref, bih_ref,
             bhh_ref, o_ref):
        rowid = lax.broadcasted_iota(jnp.int32, (NPAD, DO), 0)
        x = jnp.where(rowid < N, x_ref[...] + b3_ref[...], 0.0)

        oh = (bc_ref[...] == lax.broadcasted_iota(
            jnp.int32, (NPAD, B), 1)).astype(jnp.float32)
        ohT = (br_ref[...] == lax.broadcasted_iota(
            jnp.int32, (B, NPAD), 0)).astype(jnp.float32)

        wih = wih_ref[...]
        whh = whh_ref[...]
        bih = bih_ref[...]
        bhh = bhh_ref[...]

        h = jnp.zeros((B, DO), jnp.float32)
        c = jnp.zeros((B, DO), jnp.float32)
        q_star = jnp.zeros((B, 2 * DO), jnp.float32)
        for _ in range(STEPS):
            gates = (lax.dot_general(q_star, wih, (((1,), (1,)), ((), ())),
                                     preferred_element_type=jnp.float32)
                     + bih
                     + lax.dot_general(h, whh, (((1,), (1,)), ((), ())),
                                       preferred_element_type=jnp.float32)
                     + bhh)
            ig = jax.nn.sigmoid(gates[:, :DO])
            fg = jax.nn.sigmoid(gates[:, DO:2 * DO])
            gg = jnp.tanh(gates[:, 2 * DO:3 * DO])
            og = jax.nn.sigmoid(gates[:, 3 * DO:])
            c = fg * c + ig * gg
            h = og * jnp.tanh(c)
            q = h

            qb = jnp.dot(oh, q, preferred_element_type=jnp.float32)
            e = jnp.sum(x * qb, axis=1, keepdims=True)
            t = jnp.where(oh > 0.5, e, -1e30)
            emax = jnp.max(t, axis=0, keepdims=True)
            sub = lax.dot_general(oh, emax, (((1,), (1,)), ((), ())),
                                  preferred_element_type=jnp.float32)
            ex = jnp.exp(e - sub)
            denom = jnp.dot(ohT, ex, preferred_element_type=jnp.float32)
            dnode = jnp.dot(oh, denom, preferred_element_type=jnp.float32)
            a = ex / (dnode + 1e-16)
            r = jnp.dot(ohT, a * x, preferred_element_type=jnp.float32)
            q_star = jnp.concatenate([q, r], axis=1)
        o_ref[...] = q_star[:, :DO]

    return pl.pallas_call(
        body,
        out_shape=jax.ShapeDtypeStruct((B, DO), jnp.float32),
    )(agg3, b3, batch_col, batch_row, Wih, Whh, bih, bhh)


def kernel(node_features, edge_features, Esrc, Etgt, batch,
           W1, b1, W2, b2, W3, b3,
           ee1_Wa, ee1_ba, ee1_Wb, ee1_bb,
           ee2_Wa, ee2_ba, ee2_Wb, ee2_bb,
           ee3_Wa, ee3_ba, ee3_Wb, ee3_bb,
           lstm_Wih, lstm_Whh, lstm_bih, lstm_bhh):
    perm = jnp.argsort(Etgt)
    et_s = jnp.concatenate(
        [Etgt[perm], jnp.full((E_PAD - E,), N, jnp.int32)])
    es_s = jnp.concatenate(
        [Esrc[perm], jnp.zeros((E_PAD - E,), jnp.int32)])
    ef_s = jnp.concatenate(
        [edge_features[perm],
         jnp.zeros((E_PAD - E, DE), jnp.float32)], axis=0)
    et3 = et_s.reshape(EB, 1, EK)

    xp = jnp.zeros((NPAD, DF), jnp.float32).at[:N].set(node_features)
    batch_col = jnp.full((NPAD, 1), B, jnp.int32).at[:N, 0].set(batch)
    batch_row = batch_col.reshape(1, NPAD)

    sup1 = _mm(xp, W1)
    g1 = _sc_gather(sup1, es_s)
    agg1 = _edge_layer(ef_s, g1, et3, ee1_Wa, ee1_ba, ee1_Wb, ee1_bb, DH)
    sup2 = _mm(agg1, W2, bias=b1.reshape(1, DH), relu=True)
    g2 = _sc_gather(sup2, es_s)
    agg2 = _edge_layer(ef_s, g2, et3, ee2_Wa, ee2_ba, ee2_Wb, ee2_bb, DH)
    sup3 = _mm(agg2, W3, bias=b2.reshape(1, DH), relu=True)
    g3 = _sc_gather(sup3, es_s)
    agg3 = _edge_layer(ef_s, g3, et3, ee3_Wa, ee3_ba, ee3_Wb, ee3_bb, DO)

    return _set2set(agg3, b3.reshape(1, DO), batch_col, batch_row,
                    lstm_Wih, lstm_Whh,
                    lstm_bih.reshape(1, 4 * DO), lstm_bhh.reshape(1, 4 * DO))


# speedup vs baseline: 4.5936x; 2.1566x over previous
import functools

import jax
import jax.numpy as jnp
from jax import lax
from jax.experimental import pallas as pl
from jax.experimental.pallas import tpu as pltpu

N = 10000
E = 320000
DF = 128
DE = 16
DH = 256
DO = 128
B = 64
STEPS = 8

NPAD = 10496
EK = 4000
E_PAD = E
EB = E_PAD // EK
R = 128
RCH = 128


def _mm(x, W, bias=None, relu=False):
    n, din = x.shape
    dout = W.shape[1]
    rb = 256
    grid = n // rb

    def body(x_ref, w_ref, b_ref, o_ref):
        xv = x_ref[...]
        if relu:
            xv = jnp.maximum(xv + b_ref[...], 0.0)
        o_ref[...] = jnp.dot(xv, w_ref[...], preferred_element_type=jnp.float32)

    if bias is None:
        bias = jnp.zeros((1, din), jnp.float32)
    return pl.pallas_call(
        body,
        grid=(grid,),
        in_specs=[
            pl.BlockSpec((rb, din), lambda i: (i, 0)),
            pl.BlockSpec((din, dout), lambda i: (0, 0)),
            pl.BlockSpec((1, din), lambda i: (0, 0)),
        ],
        out_specs=pl.BlockSpec((rb, dout), lambda i: (i, 0)),
        out_shape=jax.ShapeDtypeStruct((n, dout), jnp.float32),
    )(x, W, bias)


def _edge_layer(ef_s, gathered, et3, Wa, ba, Wb, bb, dout, blk_off, nblk,
                acc=None):
    def body(*refs):
        if acc is None:
            (ef_ref, g_ref, et_s_ref, et_v_ref, wa_ref, ba_ref, wb_ref,
             bb_ref, o_ref) = refs
        else:
            (acc_ref, ef_ref, g_ref, et_s_ref, et_v_ref, wa_ref, ba_ref,
             wb_ref, bb_ref, o_ref) = refs
        i = pl.program_id(0)

        @pl.when(i == 0)
        def _init():
            if acc is None:
                o_ref[...] = jnp.zeros_like(o_ref)
            else:
                o_ref[...] = acc_ref[...]

        h = jnp.dot(ef_ref[...], wa_ref[...],
                    preferred_element_type=jnp.float32) + ba_ref[...]
        h = jnp.maximum(h, 0.0).astype(jnp.bfloat16)
        efv = jnp.dot(h, wb_ref[...].astype(jnp.bfloat16),
                      preferred_element_type=jnp.float32) + bb_ref[...]
        msg = (g_ref[...] * efv).astype(jnp.bfloat16)

        t_row = et_v_ref[0]
        t0 = et_s_ref[0, 0, 0]
        tl = et_s_ref[0, 0, EK - 1]
        base0 = (t0 // RCH) * RCH
        nch = tl // RCH - t0 // RCH + 1

        def chunk(c, carry):
            base = base0 + c * RCH
            rows = lax.broadcasted_iota(jnp.int32, (RCH, EK), 0) + base
            oht = (rows == t_row).astype(jnp.bfloat16)
            part = jnp.dot(oht, msg, preferred_element_type=jnp.float32)
            o_ref[pl.ds(base, RCH), :] += part
            return carry

        lax.fori_loop(0, nch, chunk, 0)

    dh = Wa.shape[1]
    in_specs = [
        pl.BlockSpec((EK, DE), lambda i: (i + blk_off, 0)),
        pl.BlockSpec((EK, dout), lambda i: (i, 0)),
        pl.BlockSpec((1, 1, EK), lambda i: (i + blk_off, 0, 0),
                     memory_space=pltpu.SMEM),
        pl.BlockSpec((1, 1, EK), lambda i: (i + blk_off, 0, 0)),
        pl.BlockSpec((DE, dh), lambda i: (0, 0)),
        pl.BlockSpec((1, dh), lambda i: (0, 0)),
        pl.BlockSpec((dh, dout), lambda i: (0, 0)),
        pl.BlockSpec((1, dout), lambda i: (0, 0)),
    ]
    args = [ef_s, gathered, et3, et3, Wa, ba.reshape(1, dh), Wb,
            bb.reshape(1, dout)]
    kwargs = {}
    if acc is not None:
        in_specs = [pl.BlockSpec((NPAD, dout), lambda i: (0, 0))] + in_specs
        args = [acc] + args
        kwargs["input_output_aliases"] = {0: 0}
    return pl.pallas_call(
        body,
        grid=(nblk,),
        in_specs=in_specs,
        out_specs=pl.BlockSpec((NPAD, dout), lambda i: (0, 0)),
        out_shape=jax.ShapeDtypeStruct((NPAD, dout), jnp.float32),
        **kwargs,
    )(*args)


SC_CH = 40
SC_W = 32
SC_NBUF = 5


def _sc_gather(table, idx):
    from jax.experimental.pallas import tpu_sc as plsc

    d = table.shape[1]
    n = idx.shape[0]
    per_w = n // SC_W
    SC_NCH = per_w // SC_CH
    mesh = plsc.VectorSubcoreMesh(core_axis_name="c", subcore_axis_name="s")

    @functools.partial(
        pl.kernel,
        out_type=jax.ShapeDtypeStruct((n, d), jnp.float32),
        mesh=mesh,
        scratch_types=[
            pltpu.VMEM((per_w,), jnp.int32),
            pltpu.VMEM((SC_NBUF, SC_CH, d), jnp.float32),
            [pltpu.SemaphoreType.DMA] * SC_NBUF,
            [pltpu.SemaphoreType.DMA] * SC_NBUF,
        ],
    )
    def k(table_hbm, idx_hbm, out_hbm, idx_all, rows_v, gsems, wsems):
        wid = lax.axis_index("s") * 2 + lax.axis_index("c")
        base = wid * per_w

        pltpu.sync_copy(idx_hbm.at[pl.ds(base, per_w)], idx_all)

        def g_issue(ci, b):
            pltpu.async_copy(
                table_hbm.at[idx_all.at[pl.ds(ci * SC_CH, SC_CH)]],
                rows_v.at[b], gsems[b])

        def g_wait(ci, b):
            pltpu.make_async_copy(
                table_hbm.at[idx_all.at[pl.ds(ci * SC_CH, SC_CH)]],
                rows_v.at[b], gsems[b]).wait()

        def w_issue(ci, b):
            pltpu.async_copy(rows_v.at[b],
                             out_hbm.at[pl.ds(base + ci * SC_CH, SC_CH)],
                             wsems[b])

        def w_wait(ci, b):
            pltpu.make_async_copy(rows_v.at[b],
                                  out_hbm.at[pl.ds(base + ci * SC_CH, SC_CH)],
                                  wsems[b]).wait()

        for b in range(SC_NBUF - 1):
            g_issue(b, b)

        def body(cg, carry):
            for b in range(SC_NBUF):
                ci = cg * SC_NBUF + b
                nb = (b + SC_NBUF - 1) % SC_NBUF

                g_wait(ci, b)
                w_issue(ci, b)

                @pl.when(ci + SC_NBUF - 1 < SC_NCH)
                def _():

                    @pl.when(ci - 1 >= 0)
                    def _():
                        w_wait(ci - 1, nb)

                    g_issue(ci + SC_NBUF - 1, nb)
            return carry

        lax.fori_loop(0, SC_NCH // SC_NBUF, body, 0)
        for ci in range(SC_NCH - SC_NBUF, SC_NCH):
            w_wait(ci, ci % SC_NBUF)

    return k(table, idx)


def _set2set(agg3, b3, batch_col, batch_row, Wih, Whh, bih, bhh):
    def body(x_ref, b3_ref, bc_ref, br_ref, wih_ref, whh_ref, bih_ref,
             bhh_ref, o_ref):
        rowid = lax.broadcasted_iota(jnp.int32, (NPAD, DO), 0)
        x = jnp.where(rowid < N, x_ref[...] + b3_ref[...], 0.0)

        oh = (bc_ref[...] == lax.broadcasted_iota(
            jnp.int32, (NPAD, B), 1)).astype(jnp.float32)
        ohT = (br_ref[...] == lax.broadcasted_iota(
            jnp.int32, (B, NPAD), 0)).astype(jnp.float32)

        wih = wih_ref[...]
        whh = whh_ref[...]
        bih = bih_ref[...]
        bhh = bhh_ref[...]

        h = jnp.zeros((B, DO), jnp.float32)
        c = jnp.zeros((B, DO), jnp.float32)
        q_star = jnp.zeros((B, 2 * DO), jnp.float32)
        for _ in range(STEPS):
            gates = (lax.dot_general(q_star, wih, (((1,), (1,)), ((), ())),
                                     preferred_element_type=jnp.float32)
                     + bih
                     + lax.dot_general(h, whh, (((1,), (1,)), ((), ())),
                                       preferred_element_type=jnp.float32)
                     + bhh)
            ig = jax.nn.sigmoid(gates[:, :DO])
            fg = jax.nn.sigmoid(gates[:, DO:2 * DO])
            gg = jnp.tanh(gates[:, 2 * DO:3 * DO])
            og = jax.nn.sigmoid(gates[:, 3 * DO:])
            c = fg * c + ig * gg
            h = og * jnp.tanh(c)
            q = h

            qb = jnp.dot(oh, q, preferred_element_type=jnp.float32)
            e = jnp.sum(x * qb, axis=1, keepdims=True)
            t = jnp.where(oh > 0.5, e, -1e30)
            emax = jnp.max(t, axis=0, keepdims=True)
            sub = lax.dot_general(oh, emax, (((1,), (1,)), ((), ())),
                                  preferred_element_type=jnp.float32)
            ex = jnp.exp(e - sub)
            denom = jnp.dot(ohT, ex, preferred_element_type=jnp.float32)
            dnode = jnp.dot(oh, denom, preferred_element_type=jnp.float32)
            a = ex / (dnode + 1e-16)
            r = jnp.dot(ohT, a * x, preferred_element_type=jnp.float32)
            q_star = jnp.concatenate([q, r], axis=1)
        o_ref[...] = q_star[:, :DO]

    return pl.pallas_call(
        body,
        out_shape=jax.ShapeDtypeStruct((B, DO), jnp.float32),
    )(agg3, b3, batch_col, batch_row, Wih, Whh, bih, bhh)


def kernel(node_features, edge_features, Esrc, Etgt, batch,
           W1, b1, W2, b2, W3, b3,
           ee1_Wa, ee1_ba, ee1_Wb, ee1_bb,
           ee2_Wa, ee2_ba, ee2_Wb, ee2_bb,
           ee3_Wa, ee3_ba, ee3_Wb, ee3_bb,
           lstm_Wih, lstm_Whh, lstm_bih, lstm_bhh):
    key = ((Etgt // R) << 19) | lax.iota(jnp.int32, E)
    (skey,) = lax.sort((key,), dimension=0, is_stable=False, num_keys=1)
    perm = skey & ((1 << 19) - 1)
    et_s = Etgt[perm]
    es_s = Esrc[perm]
    ef_s = edge_features[perm]
    et3 = et_s.reshape(EB, 1, EK)

    xp = jnp.zeros((NPAD, DF), jnp.float32).at[:N].set(node_features)
    batch_col = jnp.full((NPAD, 1), B, jnp.int32).at[:N, 0].set(batch)
    batch_row = batch_col.reshape(1, NPAD)

    half = E_PAD // 2
    hb = EB // 2
    es_a, es_b = es_s[:half], es_s[half:]

    def layer(x_in, W, b_in, relu_in, Wa, ba, Wb, bb, dout):
        sup = _mm(x_in, W, bias=b_in, relu=relu_in)
        ga = _sc_gather(sup, es_a)
        gb = _sc_gather(sup, es_b)
        agg_a = _edge_layer(ef_s, ga, et3, Wa, ba, Wb, bb, dout, 0, hb)
        return _edge_layer(ef_s, gb, et3, Wa, ba, Wb, bb, dout, hb, hb,
                           acc=agg_a)

    agg1 = layer(xp, W1, None, False, ee1_Wa, ee1_ba, ee1_Wb, ee1_bb, DH)
    agg2 = layer(agg1, W2, b1.reshape(1, DH), True,
                 ee2_Wa, ee2_ba, ee2_Wb, ee2_bb, DH)
    agg3 = layer(agg2, W3, b2.reshape(1, DH), True,
                 ee3_Wa, ee3_ba, ee3_Wb, ee3_bb, DO)

    return _set2set(agg3, b3.reshape(1, DO), batch_col, batch_row,
                    lstm_Wih, lstm_Whh,
                    lstm_bih.reshape(1, 4 * DO), lstm_bhh.reshape(1, 4 * DO))
